# Initial kernel scaffold; baseline (speedup 1.0000x reference)
import jax
import jax.numpy as jnp
from jax.experimental import pallas as pl


def kernel(x, W_enc, b_enc, D):
    raise NotImplementedError("write your pallas kernel here")


# speedup vs baseline: 1.9635x; 1.9635x over previous
import functools

import jax
import jax.numpy as jnp
from jax import lax
from jax.experimental import pallas as pl
from jax.experimental.pallas import tpu as pltpu
from jax.experimental.pallas import tpu_sc as plsc

DM = 768
NF = 32768
K = 64
BT = 32
FT = 2048
NT = NF // FT
LANES = 16
NCHUNK = NF // LANES

_SIGN = -(2**31)


def _sortable(bits):
    return bits ^ (lax.shift_right_arithmetic(bits, 31) & 0x7FFFFFFF)


def _enc_body(x_ref, w_ref, b_ref, pre_ref, thr_ref, cnt_ref, s_ref):
    i = pl.program_id(0)
    acc = lax.dot_general(
        x_ref[...], w_ref[...],
        dimension_numbers=(((1,), (1,)), ((), ())),
        preferred_element_type=jnp.float32,
    )
    pre = acc + b_ref[...]
    pre_ref[:, pl.ds(i * FT, FT)] = pre
    s_ref[:, pl.ds(i * FT, FT)] = _sortable(lax.bitcast_convert_type(pre, jnp.int32))

    @pl.when(i == NT - 1)
    def _():
        s = s_ref[...]

        def bit_step(t, p):
            cand = p | lax.shift_left(jnp.int32(1), jnp.int32(31) - t)
            t_s = cand ^ _SIGN
            cnt = jnp.sum((s >= t_s).astype(jnp.int32), axis=1, keepdims=True)
            return jnp.where(cnt >= K, cand, p)

        p = lax.fori_loop(0, 32, bit_step, jnp.zeros((BT, 1), jnp.int32))
        thr = p ^ _SIGN
        g = jnp.sum((s > thr).astype(jnp.int32), axis=1, keepdims=True)
        thr_ref[...] = jnp.broadcast_to(thr, (BT, 128))
        cnt_ref[...] = jnp.broadcast_to(g, (BT, 128))


def _encode_select(x, w_enc, b_enc):
    return pl.pallas_call(
        _enc_body,
        grid=(NT,),
        in_specs=[
            pl.BlockSpec((BT, DM), lambda i: (0, 0)),
            pl.BlockSpec((FT, DM), lambda i: (i, 0)),
            pl.BlockSpec((1, FT), lambda i: (0, i)),
        ],
        out_specs=[
            pl.BlockSpec((BT, NF), lambda i: (0, 0)),
            pl.BlockSpec((BT, 128), lambda i: (0, 0)),
            pl.BlockSpec((BT, 128), lambda i: (0, 0)),
        ],
        out_shape=[
            jax.ShapeDtypeStruct((BT, NF), jnp.float32),
            jax.ShapeDtypeStruct((BT, 128), jnp.int32),
            jax.ShapeDtypeStruct((BT, 128), jnp.int32),
        ],
        scratch_shapes=[pltpu.VMEM((BT, NF), jnp.int32)],
    )(x, w_enc, b_enc.reshape(1, NF))


def _sc_body(pre_hbm, thr_hbm, cnt_hbm, d_hbm,
             codes_hbm, recon_hbm,
             row_v, thr_v, g_v, idx_v, val_v, rows_v, acc_v, sem):
    b = lax.axis_index("s") * 2 + lax.axis_index("c")
    pltpu.sync_copy(pre_hbm.at[b], row_v)
    pltpu.sync_copy(thr_hbm.at[b, pl.ds(0, LANES)], thr_v)
    pltpu.sync_copy(cnt_hbm.at[b, pl.ds(0, LANES)], g_v)
    thr = thr_v[...]
    m = K - g_v[...]
    lanes = lax.broadcasted_iota(jnp.int32, (LANES,), 0)
    zeros16 = jnp.zeros((LANES,), jnp.int32)
    idx_v[pl.ds(K, LANES)] = zeros16

    def chunk(c, carry):
        run, n = carry
        v = row_v[pl.ds(c * LANES, LANES)]
        s = _sortable(lax.bitcast_convert_type(v, jnp.int32))
        gt = s > thr
        eq = s == thr
        eqi = eq.astype(jnp.int32)
        excl = plsc.cumsum(eqi) - eqi
        keep = gt | (eq & ((excl + run) < m))
        row_v[pl.ds(c * LANES, LANES)] = jnp.where(keep, v, 0.0)
        nk = plsc.all_reduce_population_count(keep)[0]
        ne = plsc.all_reduce_population_count(eq)[0]

        @pl.when(nk > 0)
        def _():
            plsc.store_compressed(idx_v.at[pl.ds(n, LANES)],
                                  c * LANES + lanes, mask=keep)
            plsc.store_compressed(val_v.at[pl.ds(n, LANES)], v, mask=keep)

        return run + ne, n + nk

    lax.fori_loop(0, NCHUNK, chunk, (jnp.int32(0), jnp.int32(0)))
    pltpu.sync_copy(row_v, codes_hbm.at[b])

    pltpu.async_copy(d_hbm.at[idx_v.at[pl.ds(0, K)]], rows_v, sem).wait()

    def col(c, _):
        def kstep(k, acc):
            vk = plsc.load_gather(val_v, [lax.broadcast(k, (LANES,))])
            return acc + vk * rows_v[k, pl.ds(c * LANES, LANES)]
        acc_v[pl.ds(c * LANES, LANES)] = lax.fori_loop(
            0, K, kstep, jnp.zeros((LANES,), jnp.float32))
        return 0
    lax.fori_loop(0, DM // LANES, col, 0)
    pltpu.sync_copy(acc_v, recon_hbm.at[b])


def _sc_decode(pre, thr, cnt, d):
    mesh = plsc.VectorSubcoreMesh(core_axis_name="c", subcore_axis_name="s")
    fn = pl.kernel(
        _sc_body,
        out_type=[
            jax.ShapeDtypeStruct((BT, NF), jnp.float32),
            jax.ShapeDtypeStruct((BT, DM), jnp.float32),
        ],
        mesh=mesh,
        scratch_types=[
            pltpu.VMEM((NF,), jnp.float32),
            pltpu.VMEM((LANES,), jnp.int32),
            pltpu.VMEM((LANES,), jnp.int32),
            pltpu.VMEM((K + LANES,), jnp.int32),
            pltpu.VMEM((K + LANES,), jnp.float32),
            pltpu.VMEM((K, DM), jnp.float32),
            pltpu.VMEM((DM,), jnp.float32),
            pltpu.SemaphoreType.DMA,
        ],
        compiler_params=pltpu.CompilerParams(needs_layout_passes=False),
    )
    return fn(pre, thr, cnt, d)


def kernel(x, W_enc, b_enc, D):
    pre, thr, cnt = _encode_select(x, W_enc, b_enc)
    codes, recon = _sc_decode(pre, thr, cnt, D)
    return (recon, codes, pre)


# speedup vs baseline: 2.7968x; 1.4244x over previous
import jax
import jax.numpy as jnp
from jax import lax
from jax.experimental import pallas as pl
from jax.experimental.pallas import tpu as pltpu
from jax.experimental.pallas import tpu_sc as plsc

DM = 768
NF = 32768
K = 64
BT = 32
FT = 2048
NT = NF // FT
LANES = 16
NCK = NF // LANES
CKT = FT // LANES

_SIGN = -(2**31)


def _sortable(bits):
    return bits ^ (lax.shift_right_arithmetic(bits, 31) & 0x7FFFFFFF)


def _enc_body(x_ref, w_ref, b_ref, pre_ref, tcf_ref, cm_ref, cms_ref):
    i = pl.program_id(0)
    acc = lax.dot_general(
        x_ref[...], w_ref[...],
        dimension_numbers=(((1,), (1,)), ((), ())),
        preferred_element_type=jnp.float32,
    )
    pre = acc + b_ref[...]
    pre_ref[:, pl.ds(i * FT, FT)] = pre
    cm = jnp.max(pre.reshape(BT, CKT, LANES), axis=2)
    cm_ref[...] = cm
    cms_ref[:, pl.ds(i * CKT, CKT)] = _sortable(
        lax.bitcast_convert_type(cm, jnp.int32))

    @pl.when(i == NT - 1)
    def _():
        s = cms_ref[...]

        def bit_step(t, p):
            cand = p | lax.shift_left(jnp.int32(1), jnp.int32(31) - t)
            t_s = cand ^ _SIGN
            cnt = jnp.sum((s >= t_s).astype(jnp.int32), axis=1, keepdims=True)
            return jnp.where(cnt >= K, cand, p)

        p = lax.fori_loop(0, 32, bit_step, jnp.zeros((BT, 1), jnp.int32))
        tcf = lax.bitcast_convert_type(_sortable(p ^ _SIGN), jnp.float32)
        tcf_ref[...] = jnp.broadcast_to(tcf, (BT, 128))


def _encode_select(x, w_enc, b_enc):
    return pl.pallas_call(
        _enc_body,
        grid=(NT,),
        in_specs=[
            pl.BlockSpec((BT, DM), lambda i: (0, 0)),
            pl.BlockSpec((FT, DM), lambda i: (i, 0)),
            pl.BlockSpec((1, FT), lambda i: (0, i)),
        ],
        out_specs=[
            pl.BlockSpec((BT, NF), lambda i: (0, 0)),
            pl.BlockSpec((BT, 128), lambda i: (0, 0)),
            pl.BlockSpec((BT, CKT), lambda i: (0, i)),
        ],
        out_shape=[
            jax.ShapeDtypeStruct((BT, NF), jnp.float32),
            jax.ShapeDtypeStruct((BT, 128), jnp.float32),
            jax.ShapeDtypeStruct((BT, NCK), jnp.float32),
        ],
        scratch_shapes=[pltpu.VMEM((BT, NCK), jnp.int32)],
    )(x, w_enc, b_enc.reshape(1, NF))


def _splat(ref, j):
    return plsc.load_gather(ref, [lax.broadcast(j, (LANES,))])


def _sc_body(pre_hbm, tcf_hbm, cmax_hbm, d_hbm,
             codes_hbm, recon_hbm,
             row_v, cmax_v, tcf_v, cid_v, idx_v, val_v, rows_v, acc_v,
             sem, semr):
    b = lax.axis_index("s") * 2 + lax.axis_index("c")
    rowcp = pltpu.async_copy(pre_hbm.at[b], row_v, semr)
    pltpu.sync_copy(cmax_hbm.at[b], cmax_v)
    pltpu.sync_copy(tcf_hbm.at[b, pl.ds(0, LANES)], tcf_v)
    tcf = tcf_v[...]
    lanes = lax.broadcasted_iota(jnp.int32, (LANES,), 0)
    idx_v[pl.ds(K, LANES)] = jnp.zeros((LANES,), jnp.int32)

    def sup(c, n):
        msk = cmax_v[pl.ds(c * LANES, LANES)] >= tcf
        plsc.store_compressed(cid_v.at[pl.ds(n, LANES)], c * LANES + lanes,
                              mask=msk)
        return n + plsc.all_reduce_population_count(msk)[0]

    ncand = lax.fori_loop(0, NCK // LANES, sup, jnp.int32(0))

    rowcp.wait()

    def conv(j, _):
        cid = _splat(cid_v, j)[0]
        v = row_v[pl.ds(cid * LANES, LANES)]
        s = _sortable(lax.bitcast_convert_type(v, jnp.int32))
        row_v[pl.ds(cid * LANES, LANES)] = plsc.bitcast(s, jnp.float32)
        return 0

    lax.fori_loop(0, ncand, conv, 0)

    def bit_step(t, p):
        candbit = p | lax.shift_left(jnp.int32(1), jnp.int32(31) - t)
        t_s = candbit ^ _SIGN

        def cnt_chunk(j, acc):
            cid = _splat(cid_v, j)[0]
            s = plsc.bitcast(row_v[pl.ds(cid * LANES, LANES)], jnp.int32)
            return acc + plsc.all_reduce_population_count(s >= t_s)[0]

        cnt = lax.fori_loop(0, ncand, cnt_chunk, jnp.int32(0))
        return jnp.where(cnt >= K, candbit, p)

    thr = lax.fori_loop(0, 32, bit_step, jnp.int32(0)) ^ _SIGN

    def gcnt(j, acc):
        cid = _splat(cid_v, j)[0]
        s = plsc.bitcast(row_v[pl.ds(cid * LANES, LANES)], jnp.int32)
        return acc + plsc.all_reduce_population_count(s > thr)[0]

    g = lax.fori_loop(0, ncand, gcnt, jnp.int32(0))
    m = K - g

    def scan(j, carry):
        run, n = carry
        cid = _splat(cid_v, j)[0]
        s = plsc.bitcast(row_v[pl.ds(cid * LANES, LANES)], jnp.int32)
        gt = s > thr
        eq = s == thr
        eqi = eq.astype(jnp.int32)
        excl = plsc.cumsum(eqi) - eqi
        keep = gt | (eq & ((excl + run) < m))
        v = lax.bitcast_convert_type(_sortable(s), jnp.float32)
        nk = plsc.all_reduce_population_count(keep)[0]
        ne = plsc.all_reduce_population_count(eq)[0]

        @pl.when(nk > 0)
        def _():
            plsc.store_compressed(idx_v.at[pl.ds(n, LANES)],
                                  cid * LANES + lanes, mask=keep)
            plsc.store_compressed(val_v.at[pl.ds(n, LANES)], v, mask=keep)

        return run + ne, n + nk

    lax.fori_loop(0, ncand, scan, (jnp.int32(0), jnp.int32(0)))

    dcp = pltpu.async_copy(d_hbm.at[idx_v.at[pl.ds(0, K)]], rows_v, sem)

    zf32 = jnp.zeros((LANES,), jnp.float32)

    def zf(c, _):
        row_v[pl.ds(c * LANES, LANES)] = zf32
        return 0

    lax.fori_loop(0, NCK, zf, 0)
    for q in range(K // LANES):
        plsc.store_scatter(row_v, [idx_v[pl.ds(q * LANES, LANES)]],
                           val_v[pl.ds(q * LANES, LANES)])
    pltpu.sync_copy(row_v, codes_hbm.at[b])

    dcp.wait()

    def col(c, _):
        def kstep(k, acc):
            vk = _splat(val_v, k)
            return acc + vk * rows_v[k, pl.ds(c * LANES, LANES)]
        acc_v[pl.ds(c * LANES, LANES)] = lax.fori_loop(
            0, K, kstep, jnp.zeros((LANES,), jnp.float32))
        return 0

    lax.fori_loop(0, DM // LANES, col, 0)
    pltpu.sync_copy(acc_v, recon_hbm.at[b])


def _sc_decode(pre, tcf, cmax, d):
    mesh = plsc.VectorSubcoreMesh(core_axis_name="c", subcore_axis_name="s")
    fn = pl.kernel(
        _sc_body,
        out_type=[
            jax.ShapeDtypeStruct((BT, NF), jnp.float32),
            jax.ShapeDtypeStruct((BT, DM), jnp.float32),
        ],
        mesh=mesh,
        scratch_types=[
            pltpu.VMEM((NF,), jnp.float32),
            pltpu.VMEM((NCK,), jnp.float32),
            pltpu.VMEM((LANES,), jnp.float32),
            pltpu.VMEM((NCK,), jnp.int32),
            pltpu.VMEM((K + LANES,), jnp.int32),
            pltpu.VMEM((K + LANES,), jnp.float32),
            pltpu.VMEM((K, DM), jnp.float32),
            pltpu.VMEM((DM,), jnp.float32),
            pltpu.SemaphoreType.DMA,
            pltpu.SemaphoreType.DMA,
        ],
        compiler_params=pltpu.CompilerParams(needs_layout_passes=False),
    )
    return fn(pre, tcf, cmax, d)


def kernel(x, W_enc, b_enc, D):
    pre, tcf, cmax = _encode_select(x, W_enc, b_enc)
    codes, recon = _sc_decode(pre, tcf, cmax, D)
    return (recon, codes, pre)


# speedup vs baseline: 3.5173x; 1.2576x over previous
import functools

import jax
import jax.numpy as jnp
from jax import lax
from jax.experimental import pallas as pl
from jax.experimental.pallas import tpu as pltpu
from jax.experimental.pallas import tpu_sc as plsc

DM = 768
NF = 32768
K = 64
BT = 32
FT = 2048
NT = NF // FT
LANES = 16
NCK = NF // LANES
CKT = FT // LANES
FP = 128

_SIGN = -(2**31)


def _sortable(bits):
    return bits ^ (lax.shift_right_arithmetic(bits, 31) & 0x7FFFFFFF)


def _enc_body(x_ref, w_ref, b_ref, pre_ref, tcf_ref, cm_ref, cms_ref):
    i = pl.program_id(0)
    acc = lax.dot_general(
        x_ref[...], w_ref[...],
        dimension_numbers=(((1,), (1,)), ((), ())),
        preferred_element_type=jnp.float32,
    )
    pre = acc + b_ref[...]
    pre_ref[:, pl.ds(i * FT, FT)] = pre
    cm = jnp.max(pre.reshape(BT, CKT, LANES), axis=2)
    cm_ref[...] = cm
    cms_ref[:, pl.ds(i * CKT, CKT)] = _sortable(
        lax.bitcast_convert_type(cm, jnp.int32))

    @pl.when(i == NT - 1)
    def _():
        s = cms_ref[...]

        def bit_step(t, p):
            cand = p | lax.shift_left(jnp.int32(1), jnp.int32(31) - t)
            t_s = cand ^ _SIGN
            cnt = jnp.sum((s >= t_s).astype(jnp.int32), axis=1, keepdims=True)
            return jnp.where(cnt >= K, cand, p)

        p = lax.fori_loop(0, 32, bit_step, jnp.zeros((BT, 1), jnp.int32))
        tcf = lax.bitcast_convert_type(_sortable(p ^ _SIGN), jnp.float32)
        tcf_ref[...] = jnp.broadcast_to(tcf, (BT, 128))


def _encode_select(x, w_enc, b_enc):
    return pl.pallas_call(
        _enc_body,
        grid=(NT,),
        in_specs=[
            pl.BlockSpec((BT, DM), lambda i: (0, 0)),
            pl.BlockSpec((FT, DM), lambda i: (i, 0)),
            pl.BlockSpec((1, FT), lambda i: (0, i)),
        ],
        out_specs=[
            pl.BlockSpec((BT, NF), lambda i: (0, 0)),
            pl.BlockSpec((BT, 128), lambda i: (0, 0)),
            pl.BlockSpec((BT, CKT), lambda i: (0, i)),
        ],
        out_shape=[
            jax.ShapeDtypeStruct((BT, NF), jnp.float32),
            jax.ShapeDtypeStruct((BT, 128), jnp.float32),
            jax.ShapeDtypeStruct((BT, NCK), jnp.float32),
        ],
        scratch_shapes=[pltpu.VMEM((BT, NCK), jnp.int32)],
    )(x, w_enc, b_enc.reshape(1, NF))


def _splat(ref, j):
    return plsc.load_gather(ref, [lax.broadcast(j, (LANES,))])


def _sc_body(pre_hbm, tcf_hbm, cmax_hbm, d_hbm,
             codes_hbm, recon_hbm,
             codes_v, cmax_v, tcf_v, cid_v, cand_v,
             idx_v, val_v, thrm_v, gm_v, rowsa_v, rowsb_v, acc_v,
             sem, semb):
    b = lax.axis_index("s") * 2 + lax.axis_index("c")
    rowcp = pltpu.async_copy(pre_hbm.at[b], codes_v, semb)
    pltpu.sync_copy(tcf_hbm.at[b, pl.ds(0, LANES)], tcf_v)
    pltpu.sync_copy(cmax_hbm.at[b], cmax_v)
    tcf = tcf_v[...]
    lanes = lax.broadcasted_iota(jnp.int32, (LANES,), 0)
    z16i = jnp.zeros((LANES,), jnp.int32)
    z16f = jnp.zeros((LANES,), jnp.float32)
    neg16 = plsc.bitcast(lax.broadcast(jnp.int32(_SIGN), (LANES,)), jnp.float32)
    idx_v[pl.ds(K, LANES)] = z16i

    def sup(c, n):
        msk = cmax_v[pl.ds(c * LANES, LANES)] >= tcf
        cidv = c * LANES + lanes
        plsc.store_compressed(cid_v.at[pl.ds(n, LANES)], cidv, mask=msk)
        return n + plsc.all_reduce_population_count(msk)[0]

    ncand = lax.fori_loop(0, NCK // LANES, sup, jnp.int32(0))
    fast = ncand <= FP
    nb = (ncand + (FP - 1)) // FP
    nch = jnp.where(fast, FP, nb * FP)

    rowcp.wait()

    def conv_one(j, cr):
        cid = _splat(cid_v, j)[0]
        v = codes_v[pl.ds(cid * LANES, LANES)]
        s = _sortable(lax.bitcast_convert_type(v, jnp.int32))
        cand_v[pl.ds(j * LANES, LANES)] = plsc.bitcast(s, jnp.float32)
        return cr

    @pl.when(fast)
    def _():
        lax.fori_loop(0, ncand, conv_one, jnp.int32(0))

        def padc(j, cr):
            cand_v[pl.ds(j * LANES, LANES)] = neg16
            return cr

        lax.fori_loop(ncand, FP, padc, jnp.int32(0))

        def count_ge(ts):
            def body(j, acc):
                s = plsc.bitcast(cand_v[pl.ds(j * LANES, LANES)], jnp.int32)
                return acc + plsc.all_reduce_population_count(s >= ts)[0]
            return plsc.parallel_loop(0, FP, 1, unroll=8,
                                      carry=jnp.int32(0))(body)

        def bit_step(t, p):
            candbit = p | lax.shift_left(jnp.int32(1), jnp.int32(31) - t)
            cnt = count_ge(candbit ^ _SIGN)
            return jnp.where(cnt >= K, candbit, p)

        thr = lax.fori_loop(0, 32, bit_step, jnp.int32(0)) ^ _SIGN

        def body_g(j, acc):
            s = plsc.bitcast(cand_v[pl.ds(j * LANES, LANES)], jnp.int32)
            return acc + plsc.all_reduce_population_count(s > thr)[0]

        g = plsc.parallel_loop(0, FP, 1, unroll=8,
                               carry=jnp.int32(0))(body_g)
        thrm_v[...] = lax.broadcast(thr, (LANES,))
        gm_v[...] = lax.broadcast(g, (LANES,))

    @pl.when(jnp.logical_not(fast))
    def _():
        lax.fori_loop(0, ncand, conv_one, jnp.int32(0))

        def padc2(j, cr):
            cand_v[pl.ds(j * LANES, LANES)] = neg16
            return cr

        lax.fori_loop(ncand, nb * FP, padc2, jnp.int32(0))

        def count_ge2(ts):
            def body(j, acc):
                s = plsc.bitcast(cand_v[pl.ds(j * LANES, LANES)], jnp.int32)
                return acc + plsc.all_reduce_population_count(s >= ts)[0]
            return lax.fori_loop(0, nb * FP, body, jnp.int32(0))

        def bit_step2(t, p):
            candbit = p | lax.shift_left(jnp.int32(1), jnp.int32(31) - t)
            cnt = count_ge2(candbit ^ _SIGN)
            return jnp.where(cnt >= K, candbit, p)

        thr = lax.fori_loop(0, 32, bit_step2, jnp.int32(0)) ^ _SIGN

        def body_g2(j, acc):
            s = plsc.bitcast(cand_v[pl.ds(j * LANES, LANES)], jnp.int32)
            return acc + plsc.all_reduce_population_count(s > thr)[0]

        g = lax.fori_loop(0, nb * FP, body_g2, jnp.int32(0))
        thrm_v[...] = lax.broadcast(thr, (LANES,))
        gm_v[...] = lax.broadcast(g, (LANES,))

    thrv = thrm_v[...]
    mv = K - gm_v[...]

    def scan(j, carry):
        run, n = carry
        s = plsc.bitcast(cand_v[pl.ds(j * LANES, LANES)], jnp.int32)
        gt = s > thrv
        eq = s == thrv
        eqi = eq.astype(jnp.int32)
        excl = plsc.cumsum(eqi) - eqi
        keep = gt | (eq & ((excl + run) < mv))
        v = lax.bitcast_convert_type(_sortable(s), jnp.float32)
        cid = _splat(cid_v, j)
        nk = plsc.all_reduce_population_count(keep)[0]
        ne = plsc.all_reduce_population_count(eq)[0]

        @pl.when(nk > 0)
        def _():
            plsc.store_compressed(idx_v.at[pl.ds(n, LANES)],
                                  cid * LANES + lanes, mask=keep)
            plsc.store_compressed(val_v.at[pl.ds(n, LANES)], v, mask=keep)

        return run + ne, n + nk

    lax.fori_loop(0, nch, scan, (jnp.int32(0), jnp.int32(0)))

    dcp1 = pltpu.async_copy(d_hbm.at[idx_v.at[pl.ds(0, K // 2)]], rowsa_v, sem)
    dcp2 = pltpu.async_copy(d_hbm.at[idx_v.at[pl.ds(K // 2, K // 2)]],
                            rowsb_v, semb)

    @plsc.parallel_loop(0, NCK, 1, unroll=8, carry=jnp.int32(0))
    def _zf(c, cr):
        codes_v[pl.ds(c * LANES, LANES)] = z16f
        return cr

    for q in range(K // LANES):
        plsc.store_scatter(codes_v, [idx_v[pl.ds(q * LANES, LANES)]],
                           val_v[pl.ds(q * LANES, LANES)])
    pltpu.sync_copy(codes_v, codes_hbm.at[b])

    @plsc.parallel_loop(0, DM // LANES, 1, unroll=8, carry=jnp.int32(0))
    def _za(c, cr):
        acc_v[pl.ds(c * LANES, LANES)] = z16f
        return cr

    dcp1.wait()

    def acc_half(rows_v, kbase):
        def kstep(k, cr):
            vk = _splat(val_v, kbase + k)

            @plsc.parallel_loop(0, DM // LANES, 1, unroll=8,
                                carry=jnp.int32(0))
            def _fma(c, cr2):
                plsc.addupdate(acc_v.at[pl.ds(c * LANES, LANES)],
                               vk * rows_v[k, pl.ds(c * LANES, LANES)])
                return cr2

            return cr

        lax.fori_loop(0, K // 2, kstep, jnp.int32(0))

    acc_half(rowsa_v, 0)
    dcp2.wait()
    acc_half(rowsb_v, K // 2)
    pltpu.sync_copy(acc_v, recon_hbm.at[b])


def _sc_decode(pre, tcf, cmax, d):
    mesh = plsc.VectorSubcoreMesh(core_axis_name="c", subcore_axis_name="s")
    fn = pl.kernel(
        _sc_body,
        out_type=[
            jax.ShapeDtypeStruct((BT, NF), jnp.float32),
            jax.ShapeDtypeStruct((BT, DM), jnp.float32),
        ],
        mesh=mesh,
        scratch_types=[
            pltpu.VMEM((NF,), jnp.float32),
            pltpu.VMEM((NCK,), jnp.float32),
            pltpu.VMEM((LANES,), jnp.float32),
            pltpu.VMEM((NCK + LANES,), jnp.int32),
            pltpu.VMEM((NF,), jnp.float32),
            pltpu.VMEM((K + LANES,), jnp.int32),
            pltpu.VMEM((K + LANES,), jnp.float32),
            pltpu.VMEM((LANES,), jnp.int32),
            pltpu.VMEM((LANES,), jnp.int32),
            pltpu.VMEM((K // 2, DM), jnp.float32),
            pltpu.VMEM((K // 2, DM), jnp.float32),
            pltpu.VMEM((DM,), jnp.float32),
            pltpu.SemaphoreType.DMA,
            pltpu.SemaphoreType.DMA,
        ],
        compiler_params=pltpu.CompilerParams(needs_layout_passes=False),
    )
    return fn(pre, tcf, cmax, d)


def kernel(x, W_enc, b_enc, D):
    pre, tcf, cmax = _encode_select(x, W_enc, b_enc)
    codes, recon = _sc_decode(pre, tcf, cmax, D)
    return (recon, codes, pre)


# speedup vs baseline: 3.6115x; 1.0268x over previous
import jax
import jax.numpy as jnp
from jax import lax
from jax.experimental import pallas as pl
from jax.experimental.pallas import tpu as pltpu
from jax.experimental.pallas import tpu_sc as plsc

DM = 768
NF = 32768
K = 64
BT = 32
FT = 2048
NT = NF // FT
LANES = 16
NCK = FT
FP = 128

_SIGN = -(2**31)


def _sortable(bits):
    return bits ^ (lax.shift_right_arithmetic(bits, 31) & 0x7FFFFFFF)


def _enc_body(x_ref, w_ref, b_ref, pre_ref, tcf_ref, cm_ref):
    i = pl.program_id(0)
    acc = lax.dot_general(
        x_ref[...], w_ref[...],
        dimension_numbers=(((1,), (1,)), ((), ())),
        preferred_element_type=jnp.float32,
    )
    pre = acc + b_ref[...]
    pre_ref[:, pl.ds(i * FT, FT)] = pre

    @pl.when(i == 0)
    def _():
        cm_ref[...] = pre

    @pl.when(i > 0)
    def _():
        cm_ref[...] = jnp.maximum(cm_ref[...], pre)

    @pl.when(i == NT - 1)
    def _():
        s = _sortable(lax.bitcast_convert_type(cm_ref[...], jnp.int32))

        def bit_step(t, p):
            cand = p | lax.shift_left(jnp.int32(1), jnp.int32(31) - t)
            t_s = cand ^ _SIGN
            cnt = jnp.sum((s >= t_s).astype(jnp.int32), axis=1, keepdims=True)
            return jnp.where(cnt >= K, cand, p)

        p = lax.fori_loop(0, 32, bit_step, jnp.zeros((BT, 1), jnp.int32))
        tcf = lax.bitcast_convert_type(_sortable(p ^ _SIGN), jnp.float32)
        tcf_ref[...] = jnp.broadcast_to(tcf, (BT, 128))


def _encode_select(x, w_enc, b_enc):
    return pl.pallas_call(
        _enc_body,
        grid=(NT,),
        in_specs=[
            pl.BlockSpec((BT, DM), lambda i: (0, 0)),
            pl.BlockSpec((FT, DM), lambda i: (i, 0)),
            pl.BlockSpec((1, FT), lambda i: (0, i)),
        ],
        out_specs=[
            pl.BlockSpec((BT, NF), lambda i: (0, 0)),
            pl.BlockSpec((BT, 128), lambda i: (0, 0)),
            pl.BlockSpec((BT, NCK), lambda i: (0, 0)),
        ],
        out_shape=[
            jax.ShapeDtypeStruct((BT, NF), jnp.float32),
            jax.ShapeDtypeStruct((BT, 128), jnp.float32),
            jax.ShapeDtypeStruct((BT, NCK), jnp.float32),
        ],
    )(x, w_enc, b_enc.reshape(1, NF))


def _splat(ref, j):
    return plsc.load_gather(ref, [lax.broadcast(j, (LANES,))])


def _sc_body(pre_hbm, tcf_hbm, cmax_hbm, d_hbm,
             codes_hbm, recon_hbm,
             codes_v, cmax_v, tcf_v, cid_v, cand_v,
             idx_v, val_v, thrm_v, gm_v, em_v, rowsa_v, rowsb_v, acc_v,
             sem, semb, semc):
    b = lax.axis_index("s") * 2 + lax.axis_index("c")
    rowcp = pltpu.async_copy(pre_hbm.at[b], codes_v, semb)
    pltpu.sync_copy(tcf_hbm.at[b, pl.ds(0, LANES)], tcf_v)
    pltpu.sync_copy(cmax_hbm.at[b], cmax_v)
    tcf = tcf_v[...]
    lanes = lax.broadcasted_iota(jnp.int32, (LANES,), 0)
    z16i = jnp.zeros((LANES,), jnp.int32)
    z16f = jnp.zeros((LANES,), jnp.float32)
    neg16 = plsc.bitcast(lax.broadcast(jnp.int32(_SIGN), (LANES,)), jnp.float32)
    idx_v[pl.ds(K, LANES)] = z16i

    def sup(c, n):
        msk = cmax_v[pl.ds(c * LANES, LANES)] >= tcf
        cidv = c * LANES + lanes
        plsc.store_compressed(cid_v.at[pl.ds(n, LANES)], cidv, mask=msk)
        return n + plsc.all_reduce_population_count(msk)[0]

    ncand = lax.fori_loop(0, NCK // LANES, sup, jnp.int32(0))
    fast = ncand <= FP
    nb = (ncand + (FP - 1)) // FP
    nch = jnp.where(fast, FP, nb * FP)

    rowcp.wait()

    def conv_one(j, cr):
        cidv16 = _splat(cid_v, j)
        v = plsc.load_gather(codes_v, [cidv16 + NCK * lanes])
        s = _sortable(lax.bitcast_convert_type(v, jnp.int32))
        cand_v[pl.ds(j * LANES, LANES)] = plsc.bitcast(s, jnp.float32)
        return cr

    def padc(j, cr):
        cand_v[pl.ds(j * LANES, LANES)] = neg16
        return cr

    @pl.when(fast)
    def _():
        lax.fori_loop(0, ncand, conv_one, jnp.int32(0))
        lax.fori_loop(ncand, FP, padc, jnp.int32(0))

        def count_ge(ts):
            def body(j, acc2):
                s = plsc.bitcast(cand_v[pl.ds(j * LANES, LANES)], jnp.int32)
                return acc2 + plsc.all_reduce_population_count(s >= ts)[0]
            return plsc.parallel_loop(0, FP, 1, unroll=8,
                                      carry=jnp.int32(0))(body)

        def bit_step(t, p):
            candbit = p | lax.shift_left(jnp.int32(1), jnp.int32(31) - t)
            cnt = count_ge(candbit ^ _SIGN)
            return jnp.where(cnt >= K, candbit, p)

        thr = lax.fori_loop(0, 32, bit_step, jnp.int32(0)) ^ _SIGN

        def body_g(j, acc2):
            s = plsc.bitcast(cand_v[pl.ds(j * LANES, LANES)], jnp.int32)
            return acc2 + plsc.all_reduce_population_count(s > thr)[0]

        g = plsc.parallel_loop(0, FP, 1, unroll=8, carry=jnp.int32(0))(body_g)
        e = count_ge(thr) - g
        thrm_v[...] = lax.broadcast(thr, (LANES,))
        gm_v[...] = lax.broadcast(g, (LANES,))
        em_v[...] = lax.broadcast(e, (LANES,))

    @pl.when(jnp.logical_not(fast))
    def _():
        lax.fori_loop(0, ncand, conv_one, jnp.int32(0))
        lax.fori_loop(ncand, nb * FP, padc, jnp.int32(0))

        def count_ge2(ts):
            def body(j, acc2):
                s = plsc.bitcast(cand_v[pl.ds(j * LANES, LANES)], jnp.int32)
                return acc2 + plsc.all_reduce_population_count(s >= ts)[0]
            return lax.fori_loop(0, nb * FP, body, jnp.int32(0))

        def bit_step2(t, p):
            candbit = p | lax.shift_left(jnp.int32(1), jnp.int32(31) - t)
            cnt = count_ge2(candbit ^ _SIGN)
            return jnp.where(cnt >= K, candbit, p)

        thr = lax.fori_loop(0, 32, bit_step2, jnp.int32(0)) ^ _SIGN

        def body_g2(j, acc2):
            s = plsc.bitcast(cand_v[pl.ds(j * LANES, LANES)], jnp.int32)
            return acc2 + plsc.all_reduce_population_count(s > thr)[0]

        g = lax.fori_loop(0, nb * FP, body_g2, jnp.int32(0))
        e = count_ge2(thr) - g
        thrm_v[...] = lax.broadcast(thr, (LANES,))
        gm_v[...] = lax.broadcast(g, (LANES,))
        em_v[...] = lax.broadcast(e, (LANES,))

    thrv = thrm_v[...]
    g0 = gm_v[...][0]
    e0 = em_v[...][0]
    m0 = K - g0

    def scan_gt(j, n):
        s = plsc.bitcast(cand_v[pl.ds(j * LANES, LANES)], jnp.int32)
        keep = s > thrv
        v = lax.bitcast_convert_type(_sortable(s), jnp.float32)
        cidv16 = _splat(cid_v, j)
        nk = plsc.all_reduce_population_count(keep)[0]

        @pl.when(nk > 0)
        def _():
            plsc.store_compressed(idx_v.at[pl.ds(n, LANES)],
                                  cidv16 + NCK * lanes, mask=keep)
            plsc.store_compressed(val_v.at[pl.ds(n, LANES)], v, mask=keep)

        return n + nk

    ng = lax.fori_loop(0, nch, scan_gt, jnp.int32(0))

    @pl.when(e0 == m0)
    def _():
        def scan_eq(j, n):
            s = plsc.bitcast(cand_v[pl.ds(j * LANES, LANES)], jnp.int32)
            keep = s == thrv
            v = lax.bitcast_convert_type(_sortable(s), jnp.float32)
            cidv16 = _splat(cid_v, j)
            nk = plsc.all_reduce_population_count(keep)[0]

            @pl.when(nk > 0)
            def _():
                plsc.store_compressed(idx_v.at[pl.ds(n, LANES)],
                                      cidv16 + NCK * lanes, mask=keep)
                plsc.store_compressed(val_v.at[pl.ds(n, LANES)], v, mask=keep)

            return n + nk

        lax.fori_loop(0, nch, scan_eq, ng)

    @pl.when(e0 != m0)
    def _():
        def jbit(t, p):
            trial = p + lax.shift_left(jnp.int32(1), jnp.int32(14) - t) - 1

            def cntj(j, acc2):
                s = plsc.bitcast(cand_v[pl.ds(j * LANES, LANES)], jnp.int32)
                gidx = _splat(cid_v, j) + NCK * lanes
                hit = (s == thrv) & (gidx <= trial)
                return acc2 + plsc.all_reduce_population_count(hit)[0]

            cnt = lax.fori_loop(0, nch, cntj, jnp.int32(0))
            return jnp.where(cnt >= m0, p,
                             p + lax.shift_left(jnp.int32(1), jnp.int32(14) - t))

        jstar = lax.fori_loop(0, 15, jbit, jnp.int32(0))

        def scan_eq2(j, n):
            s = plsc.bitcast(cand_v[pl.ds(j * LANES, LANES)], jnp.int32)
            gidx = _splat(cid_v, j) + NCK * lanes
            keep = (s == thrv) & (gidx <= jstar)
            v = lax.bitcast_convert_type(_sortable(s), jnp.float32)
            nk = plsc.all_reduce_population_count(keep)[0]

            @pl.when(nk > 0)
            def _():
                plsc.store_compressed(idx_v.at[pl.ds(n, LANES)], gidx,
                                      mask=keep)
                plsc.store_compressed(val_v.at[pl.ds(n, LANES)], v, mask=keep)

            return n + nk

        lax.fori_loop(0, nch, scan_eq2, ng)

    dcp1 = pltpu.async_copy(d_hbm.at[idx_v.at[pl.ds(0, K // 2)]], rowsa_v, sem)
    dcp2 = pltpu.async_copy(d_hbm.at[idx_v.at[pl.ds(K // 2, K // 2)]],
                            rowsb_v, semb)

    @plsc.parallel_loop(0, NF // LANES, 1, unroll=8, carry=jnp.int32(0))
    def _zf(c, cr):
        codes_v[pl.ds(c * LANES, LANES)] = z16f
        return cr

    for q in range(K // LANES):
        plsc.store_scatter(codes_v, [idx_v[pl.ds(q * LANES, LANES)]],
                           val_v[pl.ds(q * LANES, LANES)])
    ccp = pltpu.async_copy(codes_v, codes_hbm.at[b], semc)

    @plsc.parallel_loop(0, DM // LANES, 1, unroll=8, carry=jnp.int32(0))
    def _za(c, cr):
        acc_v[pl.ds(c * LANES, LANES)] = z16f
        return cr

    def acc_half(rows_v, kbase):
        def kstep(k, cr):
            vk = _splat(val_v, kbase + k)

            @plsc.parallel_loop(0, DM // LANES, 1, unroll=8,
                                carry=jnp.int32(0))
            def _fma(c, cr2):
                plsc.addupdate(acc_v.at[pl.ds(c * LANES, LANES)],
                               vk * rows_v[k, pl.ds(c * LANES, LANES)])
                return cr2

            return cr

        lax.fori_loop(0, K // 2, kstep, jnp.int32(0))

    dcp1.wait()
    acc_half(rowsa_v, 0)
    dcp2.wait()
    acc_half(rowsb_v, K // 2)
    pltpu.sync_copy(acc_v, recon_hbm.at[b])
    ccp.wait()


def _sc_decode(pre, tcf, cmax, d):
    mesh = plsc.VectorSubcoreMesh(core_axis_name="c", subcore_axis_name="s")
    fn = pl.kernel(
        _sc_body,
        out_type=[
            jax.ShapeDtypeStruct((BT, NF), jnp.float32),
            jax.ShapeDtypeStruct((BT, DM), jnp.float32),
        ],
        mesh=mesh,
        scratch_types=[
            pltpu.VMEM((NF,), jnp.float32),
            pltpu.VMEM((NCK,), jnp.float32),
            pltpu.VMEM((LANES,), jnp.float32),
            pltpu.VMEM((NCK + LANES,), jnp.int32),
            pltpu.VMEM((NF,), jnp.float32),
            pltpu.VMEM((K + LANES,), jnp.int32),
            pltpu.VMEM((K + LANES,), jnp.float32),
            pltpu.VMEM((LANES,), jnp.int32),
            pltpu.VMEM((LANES,), jnp.int32),
            pltpu.VMEM((LANES,), jnp.int32),
            pltpu.VMEM((K // 2, DM), jnp.float32),
            pltpu.VMEM((K // 2, DM), jnp.float32),
            pltpu.VMEM((DM,), jnp.float32),
            pltpu.SemaphoreType.DMA,
            pltpu.SemaphoreType.DMA,
            pltpu.SemaphoreType.DMA,
        ],
        compiler_params=pltpu.CompilerParams(needs_layout_passes=False),
    )
    return fn(pre, tcf, cmax, d)


def kernel(x, W_enc, b_enc, D):
    pre, tcf, cmax = _encode_select(x, W_enc, b_enc)
    codes, recon = _sc_decode(pre, tcf, cmax, D)
    return (recon, codes, pre)


# speedup vs baseline: 3.6542x; 1.0118x over previous
import jax
import jax.numpy as jnp
from jax import lax
from jax.experimental import pallas as pl
from jax.experimental.pallas import tpu as pltpu
from jax.experimental.pallas import tpu_sc as plsc

DM = 768
NF = 32768
K = 64
BT = 32
FT = 2048
NT = NF // FT
LANES = 16
NCK = FT
FP = 128

_SIGN = -(2**31)


def _sortable(bits):
    return bits ^ (lax.shift_right_arithmetic(bits, 31) & 0x7FFFFFFF)


def _enc_body(x_ref, w_ref, b_ref, pre_ref, tcf_ref, cm_ref):
    i = pl.program_id(0)
    acc = lax.dot_general(
        x_ref[...], w_ref[...],
        dimension_numbers=(((1,), (1,)), ((), ())),
        preferred_element_type=jnp.float32,
    )
    pre = acc + b_ref[...]
    pre_ref[:, pl.ds(i * FT, FT)] = pre

    @pl.when(i == 0)
    def _():
        cm_ref[...] = pre

    @pl.when(i > 0)
    def _():
        cm_ref[...] = jnp.maximum(cm_ref[...], pre)

    @pl.when(i == NT - 1)
    def _():
        s = _sortable(lax.bitcast_convert_type(cm_ref[...], jnp.int32))

        def bit_step(t, p):
            cand = p | lax.shift_left(jnp.int32(1), jnp.int32(31) - t)
            t_s = cand ^ _SIGN
            cnt = jnp.sum((s >= t_s).astype(jnp.int32), axis=1, keepdims=True)
            return jnp.where(cnt >= K, cand, p)

        p = lax.fori_loop(0, 32, bit_step, jnp.zeros((BT, 1), jnp.int32))
        tcf = lax.bitcast_convert_type(_sortable(p ^ _SIGN), jnp.float32)
        tcf_ref[...] = jnp.broadcast_to(tcf, (BT, 128))


def _encode_select(x, w_enc, b_enc):
    return pl.pallas_call(
        _enc_body,
        grid=(NT,),
        in_specs=[
            pl.BlockSpec((BT, DM), lambda i: (0, 0)),
            pl.BlockSpec((FT, DM), lambda i: (i, 0)),
            pl.BlockSpec((1, FT), lambda i: (0, i)),
        ],
        out_specs=[
            pl.BlockSpec((BT, NF), lambda i: (0, 0)),
            pl.BlockSpec((BT, 128), lambda i: (0, 0)),
            pl.BlockSpec((BT, NCK), lambda i: (0, 0)),
        ],
        out_shape=[
            jax.ShapeDtypeStruct((BT, NF), jnp.float32),
            jax.ShapeDtypeStruct((BT, 128), jnp.float32),
            jax.ShapeDtypeStruct((BT, NCK), jnp.float32),
        ],
    )(x, w_enc, b_enc.reshape(1, NF))


def _splat(ref, j):
    return plsc.load_gather(ref, [lax.broadcast(j, (LANES,))])


def _sc_body(pre_hbm, tcf_hbm, cmax_hbm, d_hbm,
             codes_hbm, recon_hbm,
             row_v, codes_v, cmax_v, tcf_v, cid_v, cand_v,
             idx_v, val_v, rowsa_v, rowsb_v, acc_v,
             sem, semb, semc):
    b = lax.axis_index("s") * 2 + lax.axis_index("c")
    rowcp = pltpu.async_copy(pre_hbm.at[b], row_v, semb)
    pltpu.sync_copy(tcf_hbm.at[b, pl.ds(0, LANES)], tcf_v)
    pltpu.sync_copy(cmax_hbm.at[b], cmax_v)
    tcf = tcf_v[...]
    lanes = lax.broadcasted_iota(jnp.int32, (LANES,), 0)
    z16i = jnp.zeros((LANES,), jnp.int32)
    z16f = jnp.zeros((LANES,), jnp.float32)
    neg16 = plsc.bitcast(lax.broadcast(jnp.int32(_SIGN), (LANES,)), jnp.float32)
    idx_v[pl.ds(K, LANES)] = z16i

    @plsc.parallel_loop(0, NF // LANES, 1, unroll=8, carry=jnp.int32(0))
    def _zf(c, cr):
        codes_v[pl.ds(c * LANES, LANES)] = z16f
        return cr

    def sup(c, n):
        msk = cmax_v[pl.ds(c * LANES, LANES)] >= tcf
        cidv = c * LANES + lanes
        plsc.store_compressed(cid_v.at[pl.ds(n, LANES)], cidv, mask=msk)
        return n + plsc.all_reduce_population_count(msk)[0]

    ncand = lax.fori_loop(0, NCK // LANES, sup, jnp.int32(0))
    fast = ncand <= FP

    rowcp.wait()

    def gidx_of(j):
        return _splat(cid_v, j) + NCK * lanes

    def read_row_s(j):
        v = plsc.load_gather(row_v, [gidx_of(j)])
        return _sortable(lax.bitcast_convert_type(v, jnp.int32))

    def read_cand_s(j):
        return plsc.bitcast(cand_v[pl.ds(j * LANES, LANES)], jnp.int32)

    def select_and_scan(read_s, count_loop, scan_trip):
        def count2(ts):
            def body(j, acc2):
                ge, gt = acc2
                s = read_s(j)
                return (ge + plsc.all_reduce_population_count(s >= ts)[0],
                        gt + plsc.all_reduce_population_count(s > ts)[0])
            return count_loop(body, (jnp.int32(0), jnp.int32(0)))

        def count_ge(ts):
            def body(j, acc2):
                s = read_s(j)
                return acc2 + plsc.all_reduce_population_count(s >= ts)[0]
            return count_loop(body, jnp.int32(0))

        def bit_step(t, p):
            candbit = p | lax.shift_left(jnp.int32(1), jnp.int32(31) - t)
            cnt = count_ge(candbit ^ _SIGN)
            return jnp.where(cnt >= K, candbit, p)

        thr = lax.fori_loop(0, 32, bit_step, jnp.int32(0)) ^ _SIGN
        cge, g = count2(thr)
        e = cge - g
        m = K - g
        thrv = lax.broadcast(thr, (LANES,))

        @pl.when(e == m)
        def _():
            def scan_all(j, n):
                s = read_s(j)
                keep = s >= thrv
                v = lax.bitcast_convert_type(_sortable(s), jnp.float32)
                nk = plsc.all_reduce_population_count(keep)[0]

                @pl.when(nk > 0)
                def _():
                    plsc.store_compressed(idx_v.at[pl.ds(n, LANES)],
                                          gidx_of(j), mask=keep)
                    plsc.store_compressed(val_v.at[pl.ds(n, LANES)], v,
                                          mask=keep)

                return n + nk

            lax.fori_loop(0, scan_trip, scan_all, jnp.int32(0))

        @pl.when(e != m)
        def _():
            def scan_gt(j, n):
                s = read_s(j)
                keep = s > thrv
                v = lax.bitcast_convert_type(_sortable(s), jnp.float32)
                nk = plsc.all_reduce_population_count(keep)[0]

                @pl.when(nk > 0)
                def _():
                    plsc.store_compressed(idx_v.at[pl.ds(n, LANES)],
                                          gidx_of(j), mask=keep)
                    plsc.store_compressed(val_v.at[pl.ds(n, LANES)], v,
                                          mask=keep)

                return n + nk

            ng = lax.fori_loop(0, scan_trip, scan_gt, jnp.int32(0))

            def jbit(t, p):
                step = lax.shift_left(jnp.int32(1), jnp.int32(14) - t)
                trial = p + step - 1

                def cntj(j, acc2):
                    s = read_s(j)
                    hit = (s == thrv) & (gidx_of(j) <= trial)
                    return acc2 + plsc.all_reduce_population_count(hit)[0]

                cnt = lax.fori_loop(0, scan_trip, cntj, jnp.int32(0))
                return jnp.where(cnt >= m, p, p + step)

            jstar = lax.fori_loop(0, 15, jbit, jnp.int32(0))

            def scan_eq(j, n):
                s = read_s(j)
                gidx = gidx_of(j)
                keep = (s == thrv) & (gidx <= jstar)
                v = lax.bitcast_convert_type(_sortable(s), jnp.float32)
                nk = plsc.all_reduce_population_count(keep)[0]

                @pl.when(nk > 0)
                def _():
                    plsc.store_compressed(idx_v.at[pl.ds(n, LANES)], gidx,
                                          mask=keep)
                    plsc.store_compressed(val_v.at[pl.ds(n, LANES)], v,
                                          mask=keep)

                return n + nk

            lax.fori_loop(0, scan_trip, scan_eq, ng)

    @pl.when(fast)
    def _():
        def conv_one(j, cr):
            cand_v[pl.ds(j * LANES, LANES)] = plsc.bitcast(read_row_s(j),
                                                           jnp.float32)
            return cr

        lax.fori_loop(0, ncand, conv_one, jnp.int32(0))

        def padc(j, cr):
            cand_v[pl.ds(j * LANES, LANES)] = neg16
            return cr

        lax.fori_loop(ncand, FP, padc, jnp.int32(0))

        def count_loop(body, init):
            return plsc.parallel_loop(0, FP, 1, unroll=8, carry=init)(body)

        select_and_scan(read_cand_s, count_loop, jnp.int32(FP))

    @pl.when(jnp.logical_not(fast))
    def _():
        def count_loop(body, init):
            return lax.fori_loop(0, ncand, body, init)

        select_and_scan(read_row_s, count_loop, ncand)

    dcp1 = pltpu.async_copy(d_hbm.at[idx_v.at[pl.ds(0, K // 2)]], rowsa_v, sem)
    dcp2 = pltpu.async_copy(d_hbm.at[idx_v.at[pl.ds(K // 2, K // 2)]],
                            rowsb_v, semb)

    for q in range(K // LANES):
        plsc.store_scatter(codes_v, [idx_v[pl.ds(q * LANES, LANES)]],
                           val_v[pl.ds(q * LANES, LANES)])
    ccp = pltpu.async_copy(codes_v, codes_hbm.at[b], semc)

    @plsc.parallel_loop(0, DM // LANES, 1, unroll=8, carry=jnp.int32(0))
    def _za(c, cr):
        acc_v[pl.ds(c * LANES, LANES)] = z16f
        return cr

    def acc_half(rows_v, kbase):
        def kstep(k, cr):
            vk1 = _splat(val_v, kbase + 2 * k)
            vk2 = _splat(val_v, kbase + 2 * k + 1)

            @plsc.parallel_loop(0, DM // LANES, 1, unroll=8,
                                carry=jnp.int32(0))
            def _fma(c, cr2):
                sl = pl.ds(c * LANES, LANES)
                plsc.addupdate(acc_v.at[sl],
                               vk1 * rows_v[2 * k, sl]
                               + vk2 * rows_v[2 * k + 1, sl])
                return cr2

            return cr

        lax.fori_loop(0, K // 4, kstep, jnp.int32(0))

    dcp1.wait()
    acc_half(rowsa_v, 0)
    dcp2.wait()
    acc_half(rowsb_v, K // 2)
    pltpu.sync_copy(acc_v, recon_hbm.at[b])
    ccp.wait()


def _sc_decode(pre, tcf, cmax, d):
    mesh = plsc.VectorSubcoreMesh(core_axis_name="c", subcore_axis_name="s")
    fn = pl.kernel(
        _sc_body,
        out_type=[
            jax.ShapeDtypeStruct((BT, NF), jnp.float32),
            jax.ShapeDtypeStruct((BT, DM), jnp.float32),
        ],
        mesh=mesh,
        scratch_types=[
            pltpu.VMEM((NF,), jnp.float32),
            pltpu.VMEM((NF,), jnp.float32),
            pltpu.VMEM((NCK,), jnp.float32),
            pltpu.VMEM((LANES,), jnp.float32),
            pltpu.VMEM((NCK + LANES,), jnp.int32),
            pltpu.VMEM((FP * LANES + LANES,), jnp.float32),
            pltpu.VMEM((K + LANES,), jnp.int32),
            pltpu.VMEM((K + LANES,), jnp.float32),
            pltpu.VMEM((K // 2, DM), jnp.float32),
            pltpu.VMEM((K // 2, DM), jnp.float32),
            pltpu.VMEM((DM,), jnp.float32),
            pltpu.SemaphoreType.DMA,
            pltpu.SemaphoreType.DMA,
            pltpu.SemaphoreType.DMA,
        ],
        compiler_params=pltpu.CompilerParams(needs_layout_passes=False),
    )
    return fn(pre, tcf, cmax, d)


def kernel(x, W_enc, b_enc, D):
    pre, tcf, cmax = _encode_select(x, W_enc, b_enc)
    codes, recon = _sc_decode(pre, tcf, cmax, D)
    return (recon, codes, pre)


# speedup vs baseline: 3.6840x; 1.0082x over previous
import jax
import jax.numpy as jnp
from jax import lax
from jax.experimental import pallas as pl
from jax.experimental.pallas import tpu as pltpu
from jax.experimental.pallas import tpu_sc as plsc

DM = 768
NF = 32768
K = 64
BT = 32
FT = 2048
NT = NF // FT
LANES = 16
NCK = FT
FP = 128

_SIGN = -(2**31)


def _sortable(bits):
    return bits ^ (lax.shift_right_arithmetic(bits, 31) & 0x7FFFFFFF)


def _enc_body(x_ref, w_ref, b_ref, pre_ref, tcf_ref, cm_ref):
    i = pl.program_id(0)
    acc = lax.dot_general(
        x_ref[...], w_ref[...],
        dimension_numbers=(((1,), (1,)), ((), ())),
        preferred_element_type=jnp.float32,
    )
    pre = acc + b_ref[...]
    pre_ref[:, pl.ds(i * FT, FT)] = pre

    @pl.when(i == 0)
    def _():
        cm_ref[...] = pre

    @pl.when(i > 0)
    def _():
        cm_ref[...] = jnp.maximum(cm_ref[...], pre)

    @pl.when(i == NT - 1)
    def _():
        s = _sortable(lax.bitcast_convert_type(cm_ref[...], jnp.int32))

        def bit_step(t, p):
            cand = p | lax.shift_left(jnp.int32(1), jnp.int32(31) - t)
            t_s = cand ^ _SIGN
            cnt = jnp.sum((s >= t_s).astype(jnp.int32), axis=1, keepdims=True)
            return jnp.where(cnt >= K, cand, p)

        p = lax.fori_loop(0, 32, bit_step, jnp.zeros((BT, 1), jnp.int32))
        tcf = lax.bitcast_convert_type(_sortable(p ^ _SIGN), jnp.float32)
        tcf_ref[...] = jnp.broadcast_to(tcf, (BT, 128))


def _encode_select(x, w_enc, b_enc):
    return pl.pallas_call(
        _enc_body,
        grid=(NT,),
        in_specs=[
            pl.BlockSpec((BT, DM), lambda i: (0, 0)),
            pl.BlockSpec((FT, DM), lambda i: (i, 0)),
            pl.BlockSpec((1, FT), lambda i: (0, i)),
        ],
        out_specs=[
            pl.BlockSpec((BT, NF), lambda i: (0, 0)),
            pl.BlockSpec((BT, 128), lambda i: (0, 0)),
            pl.BlockSpec((BT, NCK), lambda i: (0, 0)),
        ],
        out_shape=[
            jax.ShapeDtypeStruct((BT, NF), jnp.float32),
            jax.ShapeDtypeStruct((BT, 128), jnp.float32),
            jax.ShapeDtypeStruct((BT, NCK), jnp.float32),
        ],
    )(x, w_enc, b_enc.reshape(1, NF))


def _splat(ref, j):
    return plsc.load_gather(ref, [lax.broadcast(j, (LANES,))])


def _sc_body(pre_hbm, tcf_hbm, cmax_hbm, d_hbm,
             codes_hbm, recon_hbm,
             row_v, codes_v, cmax_v, tcf_v, cid_v, cand_v,
             idx_v, val_v, rowsa_v, rowsb_v, acc_v,
             sem, semb, semc):
    b = lax.axis_index("s") * 2 + lax.axis_index("c")
    rowcp = pltpu.async_copy(pre_hbm.at[b], row_v, semb)
    pltpu.sync_copy(tcf_hbm.at[b, pl.ds(0, LANES)], tcf_v)
    pltpu.sync_copy(cmax_hbm.at[b], cmax_v)
    tcf = tcf_v[...]
    lanes = lax.broadcasted_iota(jnp.int32, (LANES,), 0)
    z16i = jnp.zeros((LANES,), jnp.int32)
    z16f = jnp.zeros((LANES,), jnp.float32)
    neg16 = plsc.bitcast(lax.broadcast(jnp.int32(_SIGN), (LANES,)), jnp.float32)
    idx_v[pl.ds(K, LANES)] = z16i

    @plsc.parallel_loop(0, NF // LANES, 1, unroll=8, carry=jnp.int32(0))
    def _zf(c, cr):
        codes_v[pl.ds(c * LANES, LANES)] = z16f
        return cr

    def sup(c, n):
        msk = cmax_v[pl.ds(c * LANES, LANES)] >= tcf
        cidv = c * LANES + lanes
        plsc.store_compressed(cid_v.at[pl.ds(n, LANES)], cidv, mask=msk)
        return n + plsc.all_reduce_population_count(msk)[0]

    ncand = lax.fori_loop(0, NCK // LANES, sup, jnp.int32(0))
    fast = ncand <= FP

    rowcp.wait()

    def gidx_of(j):
        return _splat(cid_v, j) + NCK * lanes

    def read_row_s(j):
        v = plsc.load_gather(row_v, [gidx_of(j)])
        return _sortable(lax.bitcast_convert_type(v, jnp.int32))

    def read_cand_s(j):
        return plsc.bitcast(cand_v[pl.ds(j * LANES, LANES)], jnp.int32)

    tcs = _sortable(lax.bitcast_convert_type(tcf, jnp.int32))[0]

    def select_and_scan(read_s, count_loop, scan_trip):
        def count2(ts):
            def body(j, acc2):
                ge, gt = acc2
                s = read_s(j)
                return (ge + plsc.all_reduce_population_count(s >= ts)[0],
                        gt + plsc.all_reduce_population_count(s > ts)[0])
            return count_loop(body, (jnp.int32(0), jnp.int32(0)))

        def count_ge(ts):
            def body(j, acc2):
                s = read_s(j)
                return acc2 + plsc.all_reduce_population_count(s >= ts)[0]
            return count_loop(body, jnp.int32(0))

        def bit_step(t, p):
            candbit = p | lax.shift_left(jnp.int32(1), jnp.int32(31) - t)
            t_s = candbit ^ _SIGN
            cnt = lax.cond(t_s <= tcs, lambda: jnp.int32(K),
                           lambda: count_ge(t_s))
            return jnp.where(cnt >= K, candbit, p)

        thr = lax.fori_loop(0, 32, bit_step, jnp.int32(0)) ^ _SIGN
        cge, g = count2(thr)
        e = cge - g
        m = K - g
        thrv = lax.broadcast(thr, (LANES,))

        @pl.when(e == m)
        def _():
            def scan_all(j, n):
                s = read_s(j)
                keep = s >= thrv
                v = lax.bitcast_convert_type(_sortable(s), jnp.float32)
                nk = plsc.all_reduce_population_count(keep)[0]

                @pl.when(nk > 0)
                def _():
                    plsc.store_compressed(idx_v.at[pl.ds(n, LANES)],
                                          gidx_of(j), mask=keep)
                    plsc.store_compressed(val_v.at[pl.ds(n, LANES)], v,
                                          mask=keep)

                return n + nk

            lax.fori_loop(0, scan_trip, scan_all, jnp.int32(0))

        @pl.when(e != m)
        def _():
            def scan_gt(j, n):
                s = read_s(j)
                keep = s > thrv
                v = lax.bitcast_convert_type(_sortable(s), jnp.float32)
                nk = plsc.all_reduce_population_count(keep)[0]

                @pl.when(nk > 0)
                def _():
                    plsc.store_compressed(idx_v.at[pl.ds(n, LANES)],
                                          gidx_of(j), mask=keep)
                    plsc.store_compressed(val_v.at[pl.ds(n, LANES)], v,
                                          mask=keep)

                return n + nk

            ng = lax.fori_loop(0, scan_trip, scan_gt, jnp.int32(0))

            def jbit(t, p):
                step = lax.shift_left(jnp.int32(1), jnp.int32(14) - t)
                trial = p + step - 1

                def cntj(j, acc2):
                    s = read_s(j)
                    hit = (s == thrv) & (gidx_of(j) <= trial)
                    return acc2 + plsc.all_reduce_population_count(hit)[0]

                cnt = lax.fori_loop(0, scan_trip, cntj, jnp.int32(0))
                return jnp.where(cnt >= m, p, p + step)

            jstar = lax.fori_loop(0, 15, jbit, jnp.int32(0))

            def scan_eq(j, n):
                s = read_s(j)
                gidx = gidx_of(j)
                keep = (s == thrv) & (gidx <= jstar)
                v = lax.bitcast_convert_type(_sortable(s), jnp.float32)
                nk = plsc.all_reduce_population_count(keep)[0]

                @pl.when(nk > 0)
                def _():
                    plsc.store_compressed(idx_v.at[pl.ds(n, LANES)], gidx,
                                          mask=keep)
                    plsc.store_compressed(val_v.at[pl.ds(n, LANES)], v,
                                          mask=keep)

                return n + nk

            lax.fori_loop(0, scan_trip, scan_eq, ng)

    @pl.when(fast)
    def _():
        def conv_one(j, cr):
            cand_v[pl.ds(j * LANES, LANES)] = plsc.bitcast(read_row_s(j),
                                                           jnp.float32)
            return cr

        lax.fori_loop(0, ncand, conv_one, jnp.int32(0))

        def padc(j, cr):
            cand_v[pl.ds(j * LANES, LANES)] = neg16
            return cr

        lax.fori_loop(ncand, FP, padc, jnp.int32(0))

        def count_loop(body, init):
            return plsc.parallel_loop(0, FP, 1, unroll=8, carry=init)(body)

        select_and_scan(read_cand_s, count_loop, jnp.int32(FP))

    @pl.when(jnp.logical_not(fast))
    def _():
        def count_loop(body, init):
            return lax.fori_loop(0, ncand, body, init)

        select_and_scan(read_row_s, count_loop, ncand)

    dcp1 = pltpu.async_copy(d_hbm.at[idx_v.at[pl.ds(0, K // 2)]], rowsa_v, sem)
    dcp2 = pltpu.async_copy(d_hbm.at[idx_v.at[pl.ds(K // 2, K // 2)]],
                            rowsb_v, semb)

    for q in range(K // LANES):
        plsc.store_scatter(codes_v, [idx_v[pl.ds(q * LANES, LANES)]],
                           val_v[pl.ds(q * LANES, LANES)])
    ccp = pltpu.async_copy(codes_v, codes_hbm.at[b], semc)

    @plsc.parallel_loop(0, DM // LANES, 1, unroll=8, carry=jnp.int32(0))
    def _za(c, cr):
        acc_v[pl.ds(c * LANES, LANES)] = z16f
        return cr

    def acc_half(rows_v, kbase):
        def kstep(k, cr):
            vk1 = _splat(val_v, kbase + 2 * k)
            vk2 = _splat(val_v, kbase + 2 * k + 1)

            @plsc.parallel_loop(0, DM // LANES, 1, unroll=8,
                                carry=jnp.int32(0))
            def _fma(c, cr2):
                sl = pl.ds(c * LANES, LANES)
                plsc.addupdate(acc_v.at[sl],
                               vk1 * rows_v[2 * k, sl]
                               + vk2 * rows_v[2 * k + 1, sl])
                return cr2

            return cr

        lax.fori_loop(0, K // 4, kstep, jnp.int32(0))

    dcp1.wait()
    acc_half(rowsa_v, 0)
    dcp2.wait()
    acc_half(rowsb_v, K // 2)
    pltpu.sync_copy(acc_v, recon_hbm.at[b])
    ccp.wait()


def _sc_decode(pre, tcf, cmax, d):
    mesh = plsc.VectorSubcoreMesh(core_axis_name="c", subcore_axis_name="s")
    fn = pl.kernel(
        _sc_body,
        out_type=[
            jax.ShapeDtypeStruct((BT, NF), jnp.float32),
            jax.ShapeDtypeStruct((BT, DM), jnp.float32),
        ],
        mesh=mesh,
        scratch_types=[
            pltpu.VMEM((NF,), jnp.float32),
            pltpu.VMEM((NF,), jnp.float32),
            pltpu.VMEM((NCK,), jnp.float32),
            pltpu.VMEM((LANES,), jnp.float32),
            pltpu.VMEM((NCK + LANES,), jnp.int32),
            pltpu.VMEM((FP * LANES + LANES,), jnp.float32),
            pltpu.VMEM((K + LANES,), jnp.int32),
            pltpu.VMEM((K + LANES,), jnp.float32),
            pltpu.VMEM((K // 2, DM), jnp.float32),
            pltpu.VMEM((K // 2, DM), jnp.float32),
            pltpu.VMEM((DM,), jnp.float32),
            pltpu.SemaphoreType.DMA,
            pltpu.SemaphoreType.DMA,
            pltpu.SemaphoreType.DMA,
        ],
        compiler_params=pltpu.CompilerParams(needs_layout_passes=False),
    )
    return fn(pre, tcf, cmax, d)


def kernel(x, W_enc, b_enc, D):
    pre, tcf, cmax = _encode_select(x, W_enc, b_enc)
    codes, recon = _sc_decode(pre, tcf, cmax, D)
    return (recon, codes, pre)
